# baseline (speedup 1.0000x reference)
import functools

import jax
import jax.numpy as jnp
from jax import lax
from jax.experimental import pallas as pl
from jax.experimental.pallas import tpu as pltpu
from jax.experimental.pallas import tpu_sc as plsc

DEPTH = 3
H = 128

NC = 2
NS = 16
NW = NC * NS
CH = 128


def _init_body(fb_ref, w_ref, binput_ref, msg_ref):
    b = jnp.dot(fb_ref[...].astype(jnp.bfloat16),
                w_ref[...].astype(jnp.bfloat16),
                preferred_element_type=jnp.float32)
    binput_ref[...] = b.astype(jnp.bfloat16)
    msg_ref[...] = jnp.maximum(b, 0.0)


def _msg_update_body(nei_ref, bin_ref, w_ref, msg_ref):
    x = jnp.dot(nei_ref[...], w_ref[...], preferred_element_type=jnp.float32)
    msg_ref[...] = jnp.maximum(bin_ref[...].astype(jnp.float32) + x, 0.0)


def _atom_body(fa_ref, am_ref, wa_ref, wm_ref, b_ref, out_ref):
    h = jnp.dot(fa_ref[...], wa_ref[...], preferred_element_type=jnp.float32)
    h = h + jnp.dot(am_ref[...], wm_ref[...], preferred_element_type=jnp.float32)
    h = jnp.maximum(h + b_ref[...], 0.0)
    rows = h.shape[0]
    mols = out_ref.shape[0]
    apm = rows // mols
    r = lax.broadcasted_iota(jnp.int32, (mols, rows), 1)
    m = lax.broadcasted_iota(jnp.int32, (mols, rows), 0)
    sel = (r // apm == m).astype(jnp.float32)
    out_ref[...] = jnp.dot(sel, h, preferred_element_type=jnp.float32) * (1.0 / apm)


def _make_gather_sum(k, n_chunks, dtype=jnp.float32):
    per_w = 2 * (-(-(-(-n_chunks // NW)) // 2))
    n_pairs = per_w // 2
    mesh = plsc.VectorSubcoreMesh(core_axis_name="c", subcore_axis_name="s")

    def body(table_hbm, idx_hbm, out_hbm,
             idx_v0, idx_v1, dst_v0, dst_v1,
             sem_i0, sem_i1, sem_g0, sem_g1, sem_s0, sem_s1):
        wid = lax.axis_index("s") * NC + lax.axis_index("c")
        base = wid * per_w
        idx_v = (idx_v0, idx_v1)
        dst_v = (dst_v0, dst_v1)
        sem_i = (sem_i0, sem_i1)
        sem_g = (sem_g0, sem_g1)
        sem_s = (sem_s0, sem_s1)

        def fire_idx(c, b):
            pltpu.async_copy(idx_hbm.at[pl.ds(c * (k * CH), k * CH)],
                             idx_v[b], sem_i[b])

        def wait_idx(b):
            pltpu.make_async_copy(idx_hbm.at[pl.ds(0, k * CH)],
                                  idx_v[b], sem_i[b]).wait()

        def wait_store(b):
            pltpu.make_async_copy(dst_v[b], out_hbm.at[pl.ds(0, CH)],
                                  sem_s[b]).wait()

        for b in range(2):
            @pl.when(base + b < n_chunks)
            def _(b=b):
                fire_idx(base + b, b)

        def pair(p, carry):
            c0 = base + 2 * p
            gdescs = [[], []]

            for b in range(2):
                c = c0 + b

                @pl.when(c < n_chunks)
                def _(b=b, c=c):
                    @pl.when(p > 0)
                    def _():
                        wait_store(b)
                    wait_idx(b)

            for b in range(2):
                c = c0 + b

                @pl.when(c < n_chunks)
                def _(b=b, c=c):
                    pltpu.async_copy(
                        table_hbm.at[idx_v[b].at[pl.ds(0, CH)]],
                        dst_v[b], sem_g[b])

            for b in range(2):
                c = c0 + b

                @pl.when(c < n_chunks)
                def _(b=b, c=c):
                    pltpu.make_async_copy(
                        table_hbm.at[idx_v[b].at[pl.ds(0, CH)]],
                        dst_v[b], sem_g[b]).wait()
                    for j in range(1, k):
                        pltpu.async_copy(
                            table_hbm.at[idx_v[b].at[pl.ds(j * CH, CH)]],
                            dst_v[b], sem_g[b], add=True)

            for b in range(2):
                c = c0 + b

                @pl.when(c < n_chunks)
                def _(b=b, c=c):
                    for j in range(1, k):
                        pltpu.make_async_copy(
                            table_hbm.at[idx_v[b].at[pl.ds(j * CH, CH)]],
                            dst_v[b], sem_g[b]).wait()
                    pltpu.async_copy(dst_v[b], out_hbm.at[pl.ds(c * CH, CH)],
                                     sem_s[b])

                    @pl.when(jnp.logical_and(p + 1 < n_pairs,
                                             c + 2 < n_chunks))
                    def _():
                        fire_idx(c + 2, b)

            return carry

        lax.fori_loop(0, n_pairs, pair, 0)

        for b in range(2):
            @pl.when(base + b < n_chunks)
            def _(b=b):
                wait_store(b)

    return pl.kernel(
        body,
        out_type=jax.ShapeDtypeStruct((n_chunks * CH, H), dtype),
        mesh=mesh,
        scratch_types=[
            pltpu.VMEM((k * CH,), jnp.int32),
            pltpu.VMEM((k * CH,), jnp.int32),
            pltpu.VMEM((CH, H), dtype),
            pltpu.VMEM((CH, H), dtype),
            pltpu.SemaphoreType.DMA,
            pltpu.SemaphoreType.DMA,
            pltpu.SemaphoreType.DMA,
            pltpu.SemaphoreType.DMA,
            pltpu.SemaphoreType.DMA,
            pltpu.SemaphoreType.DMA,
        ],
    )


def _chunked_idx(idx, n_chunks):
    rows, k = idx.shape
    pad = n_chunks * CH - rows
    if pad:
        idx = jnp.pad(idx, ((0, pad), (0, 0)))
    return idx.reshape(n_chunks, CH, k).transpose(0, 2, 1).reshape(-1)


def kernel(fatoms, fbonds, agraph, bgraph, ascope, W_i, W_h, W_o_w, W_o_b):
    E, Fb = fbonds.shape
    N, Fa = fatoms.shape
    M = ascope.shape[0]
    kb = bgraph.shape[1]
    ka = agraph.shape[1]

    bgraph = bgraph.astype(jnp.int32)
    agraph = agraph.astype(jnp.int32)

    ncb = E // CH
    nca = -(-N // CH)
    idx_b = _chunked_idx(bgraph, ncb)
    idx_a = _chunked_idx(agraph, nca)

    BM1 = 3200
    binput, message = pl.pallas_call(
        _init_body,
        grid=(E // BM1,),
        in_specs=[pl.BlockSpec((BM1, Fb), lambda i: (i, 0)),
                  pl.BlockSpec((Fb, H), lambda i: (0, 0))],
        out_specs=[pl.BlockSpec((BM1, H), lambda i: (i, 0)),
                   pl.BlockSpec((BM1, H), lambda i: (i, 0))],
        out_shape=[jax.ShapeDtypeStruct((E, H), jnp.bfloat16),
                   jax.ShapeDtypeStruct((E, H), jnp.float32)],
    )(fbonds, W_i.T)

    gather_b = _make_gather_sum(kb, ncb)
    BM2 = 3200
    update = pl.pallas_call(
        _msg_update_body,
        grid=(E // BM2,),
        in_specs=[pl.BlockSpec((BM2, H), lambda i: (i, 0)),
                  pl.BlockSpec((BM2, H), lambda i: (i, 0)),
                  pl.BlockSpec((H, H), lambda i: (0, 0))],
        out_specs=pl.BlockSpec((BM2, H), lambda i: (i, 0)),
        out_shape=jax.ShapeDtypeStruct((E, H), jnp.float32),
    )
    W_hT = W_h.T
    for _ in range(DEPTH - 1):
        nei = gather_b(message, idx_b)
        message = update(nei, binput, W_hT)

    gather_a = _make_gather_sum(ka, nca)
    a_msg = gather_a(message, idx_a)[:N]

    Wa_T = W_o_w[:, :Fa].T
    Wm_T = W_o_w[:, Fa:].T
    mol_vecs = pl.pallas_call(
        _atom_body,
        grid=(1,),
        in_specs=[pl.BlockSpec((N, Fa), lambda i: (0, 0)),
                  pl.BlockSpec((N, H), lambda i: (0, 0)),
                  pl.BlockSpec((Fa, H), lambda i: (0, 0)),
                  pl.BlockSpec((H, H), lambda i: (0, 0)),
                  pl.BlockSpec((1, H), lambda i: (0, 0))],
        out_specs=pl.BlockSpec((M, H), lambda i: (0, 0)),
        out_shape=jax.ShapeDtypeStruct((M, H), jnp.float32),
    )(fatoms, a_msg, Wa_T, Wm_T, W_o_b.reshape(1, H))
    return mol_vecs


# speedup vs baseline: 1.1002x; 1.1002x over previous
import functools

import jax
import jax.numpy as jnp
from jax import lax
from jax.experimental import pallas as pl
from jax.experimental.pallas import tpu as pltpu
from jax.experimental.pallas import tpu_sc as plsc

DEPTH = 3
H = 128

NC = 2
NS = 16
NW = NC * NS
CH = 128


def _init_body(fb_ref, w_ref, binput_ref, msg_ref):
    b = jnp.dot(fb_ref[...], w_ref[...], preferred_element_type=jnp.float32)
    binput_ref[...] = b.astype(jnp.bfloat16)
    msg_ref[...] = jnp.maximum(b, 0.0)


def _msg_update_body(nei_ref, bin_ref, w_ref, msg_ref):
    x = jnp.dot(nei_ref[...], w_ref[...], preferred_element_type=jnp.float32)
    msg_ref[...] = jnp.maximum(bin_ref[...].astype(jnp.float32) + x, 0.0)


def _atom_body(fa_ref, am_ref, wa_ref, wm_ref, b_ref, out_ref):
    h = jnp.dot(fa_ref[...], wa_ref[...], preferred_element_type=jnp.float32)
    h = h + jnp.dot(am_ref[...], wm_ref[...], preferred_element_type=jnp.float32)
    h = jnp.maximum(h + b_ref[...], 0.0)
    rows = h.shape[0]
    mols = out_ref.shape[0]
    apm = rows // mols
    r = lax.broadcasted_iota(jnp.int32, (mols, rows), 1)
    m = lax.broadcasted_iota(jnp.int32, (mols, rows), 0)
    sel = (r // apm == m).astype(jnp.float32)
    out_ref[...] = jnp.dot(sel, h, preferred_element_type=jnp.float32) * (1.0 / apm)


def _make_gather_sum(k, n_chunks, ch=128, nb=2, dtype=jnp.float32):
    per_w = nb * (-(-(-(-n_chunks // NW)) // nb))
    n_groups = per_w // nb
    mesh = plsc.VectorSubcoreMesh(core_axis_name="c", subcore_axis_name="s")

    def body(table_hbm, idx_hbm, out_hbm, *scratch):
        idx_v = scratch[0:nb]
        dst_v = scratch[nb:2 * nb]
        sem_i = scratch[2 * nb:3 * nb]
        sem_g = scratch[3 * nb:4 * nb]
        sem_s = scratch[4 * nb:5 * nb]
        wid = lax.axis_index("s") * NC + lax.axis_index("c")
        base = wid * per_w

        def fire_idx(c, b):
            pltpu.async_copy(idx_hbm.at[pl.ds(c * (k * ch), k * ch)],
                             idx_v[b], sem_i[b])

        def wait_idx(b):
            pltpu.make_async_copy(idx_hbm.at[pl.ds(0, k * ch)],
                                  idx_v[b], sem_i[b]).wait()

        def wait_store(b):
            pltpu.make_async_copy(dst_v[b], out_hbm.at[pl.ds(0, ch)],
                                  sem_s[b]).wait()

        for b in range(nb):
            @pl.when(base + b < n_chunks)
            def _(b=b):
                fire_idx(base + b, b)

        def group(p, carry):
            c0 = base + nb * p

            for b in range(nb):
                c = c0 + b

                @pl.when(c < n_chunks)
                def _(b=b, c=c):
                    @pl.when(p > 0)
                    def _():
                        wait_store(b)
                    wait_idx(b)

            for b in range(nb):
                c = c0 + b

                @pl.when(c < n_chunks)
                def _(b=b, c=c):
                    pltpu.async_copy(
                        table_hbm.at[idx_v[b].at[pl.ds(0, ch)]],
                        dst_v[b], sem_g[b])

            for b in range(nb):
                c = c0 + b

                @pl.when(c < n_chunks)
                def _(b=b, c=c):
                    pltpu.make_async_copy(
                        table_hbm.at[idx_v[b].at[pl.ds(0, ch)]],
                        dst_v[b], sem_g[b]).wait()
                    for j in range(1, k):
                        pltpu.async_copy(
                            table_hbm.at[idx_v[b].at[pl.ds(j * ch, ch)]],
                            dst_v[b], sem_g[b], add=True)

            for b in range(nb):
                c = c0 + b

                @pl.when(c < n_chunks)
                def _(b=b, c=c):
                    for j in range(1, k):
                        pltpu.make_async_copy(
                            table_hbm.at[idx_v[b].at[pl.ds(j * ch, ch)]],
                            dst_v[b], sem_g[b]).wait()
                    pltpu.async_copy(dst_v[b], out_hbm.at[pl.ds(c * ch, ch)],
                                     sem_s[b])

                    @pl.when(jnp.logical_and(p + 1 < n_groups,
                                             c + nb < n_chunks))
                    def _():
                        fire_idx(c + nb, b)

            return carry

        lax.fori_loop(0, n_groups, group, 0)

        for b in range(nb):
            @pl.when(base + b < n_chunks)
            def _(b=b):
                wait_store(b)

    return pl.kernel(
        body,
        out_type=jax.ShapeDtypeStruct((n_chunks * ch, H), dtype),
        mesh=mesh,
        scratch_types=(
            [pltpu.VMEM((k * ch,), jnp.int32) for _ in range(nb)]
            + [pltpu.VMEM((ch, H), dtype) for _ in range(nb)]
            + [pltpu.SemaphoreType.DMA for _ in range(3 * nb)]
        ),
    )


def _chunked_idx(idx, n_chunks, ch=128):
    rows, k = idx.shape
    pad = n_chunks * ch - rows
    if pad:
        idx = jnp.pad(idx, ((0, pad), (0, 0)))
    return idx.reshape(n_chunks, ch, k).transpose(0, 2, 1).reshape(-1)


def kernel(fatoms, fbonds, agraph, bgraph, ascope, W_i, W_h, W_o_w, W_o_b):
    E, Fb = fbonds.shape
    N, Fa = fatoms.shape
    M = ascope.shape[0]
    kb = bgraph.shape[1]
    ka = agraph.shape[1]

    bgraph = bgraph.astype(jnp.int32)
    agraph = agraph.astype(jnp.int32)

    CHA = 64
    ncb = E // CH
    nca = -(-N // CHA)
    idx_b = _chunked_idx(bgraph, ncb, CH)
    idx_a = _chunked_idx(agraph, nca, CHA)

    BM1 = 3200
    binput, message = pl.pallas_call(
        _init_body,
        grid=(E // BM1,),
        in_specs=[pl.BlockSpec((BM1, Fb), lambda i: (i, 0)),
                  pl.BlockSpec((Fb, H), lambda i: (0, 0))],
        out_specs=[pl.BlockSpec((BM1, H), lambda i: (i, 0)),
                   pl.BlockSpec((BM1, H), lambda i: (i, 0))],
        out_shape=[jax.ShapeDtypeStruct((E, H), jnp.bfloat16),
                   jax.ShapeDtypeStruct((E, H), jnp.float32)],
    )(fbonds.astype(jnp.bfloat16), W_i.T.astype(jnp.bfloat16))

    gather_b = _make_gather_sum(kb, ncb, ch=CH, nb=4)
    BM2 = 3200
    update = pl.pallas_call(
        _msg_update_body,
        grid=(E // BM2,),
        in_specs=[pl.BlockSpec((BM2, H), lambda i: (i, 0)),
                  pl.BlockSpec((BM2, H), lambda i: (i, 0)),
                  pl.BlockSpec((H, H), lambda i: (0, 0))],
        out_specs=pl.BlockSpec((BM2, H), lambda i: (i, 0)),
        out_shape=jax.ShapeDtypeStruct((E, H), jnp.float32),
    )
    W_hT = W_h.T
    for _ in range(DEPTH - 1):
        nei = gather_b(message, idx_b)
        message = update(nei, binput, W_hT)

    gather_a = _make_gather_sum(ka, nca, ch=CHA, nb=2)
    a_msg = gather_a(message, idx_a)[:N]

    Wa_T = W_o_w[:, :Fa].T
    Wm_T = W_o_w[:, Fa:].T
    mol_vecs = pl.pallas_call(
        _atom_body,
        grid=(1,),
        in_specs=[pl.BlockSpec((N, Fa), lambda i: (0, 0)),
                  pl.BlockSpec((N, H), lambda i: (0, 0)),
                  pl.BlockSpec((Fa, H), lambda i: (0, 0)),
                  pl.BlockSpec((H, H), lambda i: (0, 0)),
                  pl.BlockSpec((1, H), lambda i: (0, 0))],
        out_specs=pl.BlockSpec((M, H), lambda i: (0, 0)),
        out_shape=jax.ShapeDtypeStruct((M, H), jnp.float32),
    )(fatoms, a_msg, Wa_T, Wm_T, W_o_b.reshape(1, H))
    return mol_vecs


# speedup vs baseline: 1.1218x; 1.0197x over previous
import functools

import jax
import jax.numpy as jnp
from jax import lax
from jax.experimental import pallas as pl
from jax.experimental.pallas import tpu as pltpu
from jax.experimental.pallas import tpu_sc as plsc

DEPTH = 3
H = 128

NC = 2
NS = 16
NW = NC * NS
CH = 128


def _init_body(fb_ref, w_ref, binput_ref, msg_ref):
    b = jnp.dot(fb_ref[...], w_ref[...], preferred_element_type=jnp.float32)
    binput_ref[...] = b.astype(jnp.bfloat16)
    msg_ref[...] = jnp.maximum(b, 0.0)


def _msg_update_body(nei_ref, bin_ref, w_ref, msg_ref):
    x = jnp.dot(nei_ref[...], w_ref[...], preferred_element_type=jnp.float32)
    msg_ref[...] = jnp.maximum(bin_ref[...].astype(jnp.float32) + x, 0.0)


def _atom_body(fa_ref, am_ref, wa_ref, wm_ref, b_ref, out_ref):
    h = jnp.dot(fa_ref[...], wa_ref[...], preferred_element_type=jnp.float32)
    h = h + jnp.dot(am_ref[...], wm_ref[...], preferred_element_type=jnp.float32)
    h = jnp.maximum(h + b_ref[...], 0.0)
    rows = h.shape[0]
    mols = out_ref.shape[0]
    apm = rows // mols
    r = lax.broadcasted_iota(jnp.int32, (mols, rows), 1)
    m = lax.broadcasted_iota(jnp.int32, (mols, rows), 0)
    sel = (r // apm == m).astype(jnp.float32)
    out_ref[...] = jnp.dot(sel, h, preferred_element_type=jnp.float32) * (1.0 / apm)


def _make_gather_sum(k, n_chunks, ch=128, nb=2, dtype=jnp.float32):
    per_w = nb * (-(-(-(-n_chunks // NW)) // nb))
    n_groups = per_w // nb
    mesh = plsc.VectorSubcoreMesh(core_axis_name="c", subcore_axis_name="s")

    def body(table_hbm, idx_hbm, out_hbm, *scratch):
        idx_v = scratch[0:nb]
        dst_v = scratch[nb:2 * nb]
        sem_i = scratch[2 * nb:3 * nb]
        sem_g = scratch[3 * nb:4 * nb]
        sem_s = scratch[4 * nb:5 * nb]
        wid = lax.axis_index("s") * NC + lax.axis_index("c")
        base = wid * per_w

        def fire_idx(c, b):
            pltpu.async_copy(idx_hbm.at[pl.ds(c * (k * ch), k * ch)],
                             idx_v[b], sem_i[b])

        def wait_idx(b):
            pltpu.make_async_copy(idx_hbm.at[pl.ds(0, k * ch)],
                                  idx_v[b], sem_i[b]).wait()

        def wait_store(b):
            pltpu.make_async_copy(dst_v[b], out_hbm.at[pl.ds(0, ch)],
                                  sem_s[b]).wait()

        for b in range(nb):
            @pl.when(base + b < n_chunks)
            def _(b=b):
                fire_idx(base + b, b)

        def group(p, carry):
            c0 = base + nb * p

            for b in range(nb):
                c = c0 + b

                @pl.when(c < n_chunks)
                def _(b=b, c=c):
                    @pl.when(p > 0)
                    def _():
                        wait_store(b)
                    wait_idx(b)

            for b in range(nb):
                c = c0 + b

                @pl.when(c < n_chunks)
                def _(b=b, c=c):
                    pltpu.async_copy(
                        table_hbm.at[idx_v[b].at[pl.ds(0, ch)]],
                        dst_v[b], sem_g[b])

            for b in range(nb):
                c = c0 + b

                @pl.when(c < n_chunks)
                def _(b=b, c=c):
                    pltpu.make_async_copy(
                        table_hbm.at[idx_v[b].at[pl.ds(0, ch)]],
                        dst_v[b], sem_g[b]).wait()
                    for j in range(1, k):
                        pltpu.async_copy(
                            table_hbm.at[idx_v[b].at[pl.ds(j * ch, ch)]],
                            dst_v[b], sem_g[b], add=True)

            for b in range(nb):
                c = c0 + b

                @pl.when(c < n_chunks)
                def _(b=b, c=c):
                    for j in range(1, k):
                        pltpu.make_async_copy(
                            table_hbm.at[idx_v[b].at[pl.ds(j * ch, ch)]],
                            dst_v[b], sem_g[b]).wait()
                    pltpu.async_copy(dst_v[b], out_hbm.at[pl.ds(c * ch, ch)],
                                     sem_s[b])

                    @pl.when(jnp.logical_and(p + 1 < n_groups,
                                             c + nb < n_chunks))
                    def _():
                        fire_idx(c + nb, b)

            return carry

        lax.fori_loop(0, n_groups, group, 0)

        for b in range(nb):
            @pl.when(base + b < n_chunks)
            def _(b=b):
                wait_store(b)

    return pl.kernel(
        body,
        out_type=jax.ShapeDtypeStruct((n_chunks * ch, H), dtype),
        mesh=mesh,
        scratch_types=(
            [pltpu.VMEM((k * ch,), jnp.int32) for _ in range(nb)]
            + [pltpu.VMEM((ch, H), dtype) for _ in range(nb)]
            + [pltpu.SemaphoreType.DMA for _ in range(3 * nb)]
        ),
    )


def _chunked_idx(idx, n_chunks, ch=128):
    rows, k = idx.shape
    pad = n_chunks * ch - rows
    if pad:
        idx = jnp.pad(idx, ((0, pad), (0, 0)))
    return idx.reshape(n_chunks, ch, k).transpose(0, 2, 1).reshape(-1)


def kernel(fatoms, fbonds, agraph, bgraph, ascope, W_i, W_h, W_o_w, W_o_b):
    E, Fb = fbonds.shape
    N, Fa = fatoms.shape
    M = ascope.shape[0]
    kb = bgraph.shape[1]
    ka = agraph.shape[1]

    bgraph = bgraph.astype(jnp.int32)
    agraph = agraph.astype(jnp.int32)

    CHA = 64
    ncb = E // CH
    nca = -(-N // CHA)
    idx_b = _chunked_idx(bgraph, ncb, CH)
    idx_a = _chunked_idx(agraph, nca, CHA)

    BM1 = 3200
    binput, message = pl.pallas_call(
        _init_body,
        grid=(E // BM1,),
        in_specs=[pl.BlockSpec((BM1, Fb), lambda i: (i, 0)),
                  pl.BlockSpec((Fb, H), lambda i: (0, 0))],
        out_specs=[pl.BlockSpec((BM1, H), lambda i: (i, 0)),
                   pl.BlockSpec((BM1, H), lambda i: (i, 0))],
        out_shape=[jax.ShapeDtypeStruct((E, H), jnp.bfloat16),
                   jax.ShapeDtypeStruct((E, H), jnp.float32)],
    )(fbonds.astype(jnp.bfloat16), W_i.T.astype(jnp.bfloat16))

    gather_b = _make_gather_sum(kb, ncb, ch=CH, nb=6)
    BM2 = 3200
    update = pl.pallas_call(
        _msg_update_body,
        grid=(E // BM2,),
        in_specs=[pl.BlockSpec((BM2, H), lambda i: (i, 0)),
                  pl.BlockSpec((BM2, H), lambda i: (i, 0)),
                  pl.BlockSpec((H, H), lambda i: (0, 0))],
        out_specs=pl.BlockSpec((BM2, H), lambda i: (i, 0)),
        out_shape=jax.ShapeDtypeStruct((E, H), jnp.float32),
    )
    W_hT = W_h.T
    for _ in range(DEPTH - 1):
        nei = gather_b(message, idx_b)
        message = update(nei, binput, W_hT)

    gather_a = _make_gather_sum(ka, nca, ch=CHA, nb=2)
    a_msg = gather_a(message, idx_a)[:N]

    Wa_T = W_o_w[:, :Fa].T
    Wm_T = W_o_w[:, Fa:].T
    mol_vecs = pl.pallas_call(
        _atom_body,
        grid=(1,),
        in_specs=[pl.BlockSpec((N, Fa), lambda i: (0, 0)),
                  pl.BlockSpec((N, H), lambda i: (0, 0)),
                  pl.BlockSpec((Fa, H), lambda i: (0, 0)),
                  pl.BlockSpec((H, H), lambda i: (0, 0)),
                  pl.BlockSpec((1, H), lambda i: (0, 0))],
        out_specs=pl.BlockSpec((M, H), lambda i: (0, 0)),
        out_shape=jax.ShapeDtypeStruct((M, H), jnp.float32),
    )(fatoms, a_msg, Wa_T, Wm_T, W_o_b.reshape(1, H))
    return mol_vecs
